# baseline (speedup 1.0000x reference)
import jax
import jax.numpy as jnp
from jax.experimental import pallas as pl
from jax.experimental.pallas import tpu as pltpu

MODEL_DIM = 2048
NUM_EXPERTS = 8
NUM_TOKENS = 4096
CAPACITY = 512
TOKEN_BLOCK = 256
GRID = NUM_TOKENS // TOKEN_BLOCK


def _router_kernel(x_ref, w_ref, laux_ref, combine_ref, dispatch_ref,
                   counts_ref, gsum_ref):
    i = pl.program_id(0)

    @pl.when(i == 0)
    def _init():
        counts_ref[...] = jnp.zeros_like(counts_ref)
        gsum_ref[...] = jnp.zeros_like(gsum_ref)

    x = x_ref[...]
    w = w_ref[...]
    logits = jax.lax.dot_general(
        x, w, (((1,), (1,)), ((), ())), preferred_element_type=jnp.float32)
    m = jnp.max(logits, axis=1, keepdims=True)
    e = jnp.exp(logits - m)
    s = jnp.sum(e, axis=1, keepdims=True)
    gates = e / s

    eids = jax.lax.broadcasted_iota(jnp.int32, (TOKEN_BLOCK, NUM_EXPERTS), 1)
    gmax = jnp.max(gates, axis=1, keepdims=True)
    idx = jnp.min(jnp.where(gates == gmax, eids, NUM_EXPERTS),
                  axis=1, keepdims=True)
    mask = (eids == idx).astype(jnp.float32)

    r = jax.lax.broadcasted_iota(jnp.int32, (TOKEN_BLOCK, TOKEN_BLOCK), 0)
    c = jax.lax.broadcasted_iota(jnp.int32, (TOKEN_BLOCK, TOKEN_BLOCK), 1)
    tri = (c <= r).astype(jnp.float32)
    cum = jax.lax.dot_general(
        tri, mask, (((1,), (0,)), ((), ())), preferred_element_type=jnp.float32)

    base = counts_ref[...]
    loc = cum + base - 1.0
    loc_tok = jnp.sum(loc * mask, axis=1, keepdims=True)
    keep = loc_tok < float(CAPACITY)
    target = jnp.where(keep, idx * CAPACITY + loc_tok.astype(jnp.int32), -1)

    col = jax.lax.broadcasted_iota(
        jnp.int32, (TOKEN_BLOCK, NUM_EXPERTS * CAPACITY), 1)
    hit = col == target
    combine_ref[...] = jnp.where(hit, gmax, 0.0)
    dispatch_ref[...] = hit

    counts_ref[...] = base + jnp.sum(mask, axis=0, keepdims=True)
    gsum_ref[...] = gsum_ref[...] + jnp.sum(gates, axis=0, keepdims=True)

    @pl.when(i == GRID - 1)
    def _finalize():
        me_ce = jnp.sum(counts_ref[...] * gsum_ref[...], keepdims=True)
        n = float(NUM_TOKENS)
        laux_ref[...] = me_ce / (n * n) * float(NUM_EXPERTS)


def kernel(input, wg_weight):
    laux, combine2d, dispatch2d = pl.pallas_call(
        _router_kernel,
        grid=(GRID,),
        in_specs=[
            pl.BlockSpec((TOKEN_BLOCK, MODEL_DIM), lambda i: (i, 0)),
            pl.BlockSpec((NUM_EXPERTS, MODEL_DIM), lambda i: (0, 0)),
        ],
        out_specs=[
            pl.BlockSpec((1, 1), lambda i: (0, 0)),
            pl.BlockSpec((TOKEN_BLOCK, NUM_EXPERTS * CAPACITY), lambda i: (i, 0)),
            pl.BlockSpec((TOKEN_BLOCK, NUM_EXPERTS * CAPACITY), lambda i: (i, 0)),
        ],
        out_shape=[
            jax.ShapeDtypeStruct((1, 1), jnp.float32),
            jax.ShapeDtypeStruct((NUM_TOKENS, NUM_EXPERTS * CAPACITY), jnp.float32),
            jax.ShapeDtypeStruct((NUM_TOKENS, NUM_EXPERTS * CAPACITY), jnp.bool_),
        ],
        scratch_shapes=[
            pltpu.VMEM((1, NUM_EXPERTS), jnp.float32),
            pltpu.VMEM((1, NUM_EXPERTS), jnp.float32),
        ],
        compiler_params=pltpu.CompilerParams(
            dimension_semantics=("arbitrary",),
        ),
    )(input, wg_weight)
    combine = combine2d.reshape(NUM_TOKENS, NUM_EXPERTS, CAPACITY)
    dispatch = dispatch2d.reshape(NUM_TOKENS, NUM_EXPERTS, CAPACITY)
    return (laux[0, 0], combine, dispatch)


# speedup vs baseline: 2.0708x; 2.0708x over previous
import jax
import jax.numpy as jnp
from jax.experimental import pallas as pl
from jax.experimental.pallas import tpu as pltpu

MODEL_DIM = 2048
NUM_EXPERTS = 8
NUM_TOKENS = 4096
CAPACITY = 512
TOKEN_BLOCK = 256
GRID = NUM_TOKENS // TOKEN_BLOCK


def _router_kernel(x_ref, w_ref, laux_ref, combine_ref, dispatch_ref,
                   counts_ref, gsum_ref):
    i = pl.program_id(0)

    @pl.when(i == 0)
    def _init():
        counts_ref[...] = jnp.zeros_like(counts_ref)
        gsum_ref[...] = jnp.zeros_like(gsum_ref)

    x = x_ref[...]
    w = w_ref[...]
    logits = jax.lax.dot_general(
        x, w, (((1,), (1,)), ((), ())), preferred_element_type=jnp.float32)
    m = jnp.max(logits, axis=1, keepdims=True)
    e = jnp.exp(logits - m)
    s = jnp.sum(e, axis=1, keepdims=True)
    gates = e / s

    eids = jax.lax.broadcasted_iota(jnp.int32, (TOKEN_BLOCK, NUM_EXPERTS), 1)
    gmax = jnp.max(gates, axis=1, keepdims=True)
    idx = jnp.min(jnp.where(gates == gmax, eids, NUM_EXPERTS),
                  axis=1, keepdims=True)
    mask = (eids == idx).astype(jnp.float32)

    r = jax.lax.broadcasted_iota(jnp.int32, (TOKEN_BLOCK, TOKEN_BLOCK), 0)
    c = jax.lax.broadcasted_iota(jnp.int32, (TOKEN_BLOCK, TOKEN_BLOCK), 1)
    tri = (c <= r).astype(jnp.float32)
    cum = jax.lax.dot_general(
        tri, mask, (((1,), (0,)), ((), ())), preferred_element_type=jnp.float32)

    base = counts_ref[...]
    loc = cum + base - 1.0
    loc_tok = jnp.sum(loc * mask, axis=1, keepdims=True)
    keep = loc_tok < float(CAPACITY)
    target = jnp.where(keep, idx * CAPACITY + loc_tok.astype(jnp.int32), -1)

    e3 = jax.lax.broadcasted_iota(
        jnp.int32, (TOKEN_BLOCK, NUM_EXPERTS, CAPACITY), 1)
    c3 = jax.lax.broadcasted_iota(
        jnp.int32, (TOKEN_BLOCK, NUM_EXPERTS, CAPACITY), 2)
    col3 = e3 * CAPACITY + c3
    hit = col3 == target[:, :, None]
    combine_ref[...] = jnp.where(hit, gmax[:, :, None], 0.0)
    dispatch_ref[...] = hit

    counts_ref[...] = base + jnp.sum(mask, axis=0, keepdims=True)
    gsum_ref[...] = gsum_ref[...] + jnp.sum(gates, axis=0, keepdims=True)

    @pl.when(i == GRID - 1)
    def _finalize():
        me_ce = jnp.sum(counts_ref[...] * gsum_ref[...], keepdims=True)
        n = float(NUM_TOKENS)
        laux_ref[...] = me_ce / (n * n) * float(NUM_EXPERTS)


def kernel(input, wg_weight):
    laux, combine, dispatch = pl.pallas_call(
        _router_kernel,
        grid=(GRID,),
        in_specs=[
            pl.BlockSpec((TOKEN_BLOCK, MODEL_DIM), lambda i: (i, 0)),
            pl.BlockSpec((NUM_EXPERTS, MODEL_DIM), lambda i: (0, 0)),
        ],
        out_specs=[
            pl.BlockSpec((1, 1), lambda i: (0, 0)),
            pl.BlockSpec((TOKEN_BLOCK, NUM_EXPERTS, CAPACITY),
                         lambda i: (i, 0, 0)),
            pl.BlockSpec((TOKEN_BLOCK, NUM_EXPERTS, CAPACITY),
                         lambda i: (i, 0, 0)),
        ],
        out_shape=[
            jax.ShapeDtypeStruct((1, 1), jnp.float32),
            jax.ShapeDtypeStruct((NUM_TOKENS, NUM_EXPERTS, CAPACITY),
                                 jnp.float32),
            jax.ShapeDtypeStruct((NUM_TOKENS, NUM_EXPERTS, CAPACITY),
                                 jnp.bool_),
        ],
        scratch_shapes=[
            pltpu.VMEM((1, NUM_EXPERTS), jnp.float32),
            pltpu.VMEM((1, NUM_EXPERTS), jnp.float32),
        ],
        compiler_params=pltpu.CompilerParams(
            dimension_semantics=("arbitrary",),
        ),
    )(input, wg_weight)
    return (laux[0, 0], combine, dispatch)


# speedup vs baseline: 2.1107x; 1.0193x over previous
import jax
import jax.numpy as jnp
from jax.experimental import pallas as pl
from jax.experimental.pallas import tpu as pltpu

MODEL_DIM = 2048
NUM_EXPERTS = 8
NUM_TOKENS = 4096
CAPACITY = 512
TOKEN_BLOCK = 512
GRID = NUM_TOKENS // TOKEN_BLOCK


def _router_kernel(x_ref, w_ref, laux_ref, combine_ref, dispatch_ref,
                   counts_ref, gsum_ref):
    i = pl.program_id(0)

    @pl.when(i == 0)
    def _init():
        counts_ref[...] = jnp.zeros_like(counts_ref)
        gsum_ref[...] = jnp.zeros_like(gsum_ref)

    x = x_ref[...]
    w = w_ref[...]
    logits = jax.lax.dot_general(
        x, w, (((1,), (1,)), ((), ())), preferred_element_type=jnp.float32)
    m = jnp.max(logits, axis=1, keepdims=True)
    e = jnp.exp(logits - m)
    s = jnp.sum(e, axis=1, keepdims=True)
    gates = e / s

    eids = jax.lax.broadcasted_iota(jnp.int32, (TOKEN_BLOCK, NUM_EXPERTS), 1)
    gmax = jnp.max(gates, axis=1, keepdims=True)
    idx = jnp.min(jnp.where(gates == gmax, eids, NUM_EXPERTS),
                  axis=1, keepdims=True)
    mask = (eids == idx).astype(jnp.float32)

    r = jax.lax.broadcasted_iota(jnp.int32, (TOKEN_BLOCK, TOKEN_BLOCK), 0)
    c = jax.lax.broadcasted_iota(jnp.int32, (TOKEN_BLOCK, TOKEN_BLOCK), 1)
    tri = (c <= r).astype(jnp.float32)
    cum = jax.lax.dot_general(
        tri, mask, (((1,), (0,)), ((), ())), preferred_element_type=jnp.float32)

    base = counts_ref[...]
    loc = cum + base - 1.0
    loc_tok = jnp.sum(loc * mask, axis=1, keepdims=True)
    keep = loc_tok < float(CAPACITY)
    target = jnp.where(keep, idx * CAPACITY + loc_tok.astype(jnp.int32), -1)

    e3 = jax.lax.broadcasted_iota(
        jnp.int32, (TOKEN_BLOCK, NUM_EXPERTS, CAPACITY), 1)
    c3 = jax.lax.broadcasted_iota(
        jnp.int32, (TOKEN_BLOCK, NUM_EXPERTS, CAPACITY), 2)
    col3 = e3 * CAPACITY + c3
    hit = col3 == target[:, :, None]
    combine_ref[...] = jnp.where(hit, gmax[:, :, None], 0.0)
    dispatch_ref[...] = hit

    counts_ref[...] = base + jnp.sum(mask, axis=0, keepdims=True)
    gsum_ref[...] = gsum_ref[...] + jnp.sum(gates, axis=0, keepdims=True)

    @pl.when(i == GRID - 1)
    def _finalize():
        me_ce = jnp.sum(counts_ref[...] * gsum_ref[...], keepdims=True)
        n = float(NUM_TOKENS)
        laux_ref[...] = me_ce / (n * n) * float(NUM_EXPERTS)


def kernel(input, wg_weight):
    laux, combine, dispatch = pl.pallas_call(
        _router_kernel,
        grid=(GRID,),
        in_specs=[
            pl.BlockSpec((TOKEN_BLOCK, MODEL_DIM), lambda i: (i, 0)),
            pl.BlockSpec((NUM_EXPERTS, MODEL_DIM), lambda i: (0, 0)),
        ],
        out_specs=[
            pl.BlockSpec((1, 1), lambda i: (0, 0)),
            pl.BlockSpec((TOKEN_BLOCK, NUM_EXPERTS, CAPACITY),
                         lambda i: (i, 0, 0)),
            pl.BlockSpec((TOKEN_BLOCK, NUM_EXPERTS, CAPACITY),
                         lambda i: (i, 0, 0)),
        ],
        out_shape=[
            jax.ShapeDtypeStruct((1, 1), jnp.float32),
            jax.ShapeDtypeStruct((NUM_TOKENS, NUM_EXPERTS, CAPACITY),
                                 jnp.float32),
            jax.ShapeDtypeStruct((NUM_TOKENS, NUM_EXPERTS, CAPACITY),
                                 jnp.bool_),
        ],
        scratch_shapes=[
            pltpu.VMEM((1, NUM_EXPERTS), jnp.float32),
            pltpu.VMEM((1, NUM_EXPERTS), jnp.float32),
        ],
        compiler_params=pltpu.CompilerParams(
            dimension_semantics=("arbitrary",),
        ),
    )(input, wg_weight)
    return (laux[0, 0], combine, dispatch)
